# Initial kernel scaffold; baseline (speedup 1.0000x reference)
import jax
import jax.numpy as jnp
from jax.experimental import pallas as pl


def kernel(x, edge_index, batch_index, W1, b1, W2, b2, Wfc, bfc):
    raise NotImplementedError("write your pallas kernel here")


# speedup vs baseline: 16.2775x; 16.2775x over previous
import functools

import jax
import jax.numpy as jnp
from jax import lax
from jax.experimental import pallas as pl
from jax.experimental.pallas import tpu as pltpu
from jax.experimental.pallas import tpu_sc as plsc

_NC = 2
_NS = 16
_G = 64
_K = 80


def _sc_mesh():
    return plsc.VectorSubcoreMesh(core_axis_name="c", subcore_axis_name="s")


def _sc_degree(dst, zeros_n):
    (n_nodes,) = zeros_n.shape
    (n_edges,) = dst.shape
    epw = n_edges // (_NC * _NS)
    chunks = epw // _K

    @functools.partial(
        pl.kernel,
        mesh=_sc_mesh(),
        out_type=jax.ShapeDtypeStruct((_NC * n_nodes,), jnp.float32),
        scratch_types=[
            pltpu.VMEM((1, _K), jnp.int32),
            pltpu.VMEM((_K,), jnp.float32),
            pltpu.VMEM_SHARED((n_nodes,), jnp.float32),
            pltpu.VMEM((n_nodes,), jnp.float32),
        ],
    )
    def deg_kernel(dst_hbm, zeros_hbm, out_hbm, idxb, onesb, acc, wb):
        c = lax.axis_index("c")
        s = lax.axis_index("s")
        for j in range(_K // 16):
            onesb[pl.ds(j * 16, 16)] = jnp.ones((16,), jnp.float32)

        @pl.when(s == 0)
        def _():
            pltpu.sync_copy(zeros_hbm, wb)
            pltpu.sync_copy(wb, acc)

        plsc.subcore_barrier()

        base = (c * _NS + s) * epw

        def body(i, carry):
            off = pl.multiple_of(base + i * _K, 8)
            pltpu.sync_copy(dst_hbm.at[pl.ds(off, _K)], idxb.at[0])
            pltpu.sync_copy(onesb, acc.at[idxb.at[0]], add=True)
            return carry

        lax.fori_loop(0, chunks, body, 0)
        plsc.subcore_barrier()

        @pl.when(s == 0)
        def _():
            pltpu.sync_copy(acc, wb)
            pltpu.sync_copy(wb, out_hbm.at[pl.ds(pl.multiple_of(c * n_nodes, 8), n_nodes)])

    return deg_kernel(dst, zeros_n)


def _sc_scatter(src, dst, rows_tbl, zeros_nf):
    (n_edges,) = src.shape
    n_nodes, feat = rows_tbl.shape
    epw = n_edges // (_NC * _NS)
    chunks = epw // _K
    rpt = (n_nodes // _NS) // 8 * 8
    tail = n_nodes - rpt * _NS

    @functools.partial(
        pl.kernel,
        mesh=_sc_mesh(),
        compiler_params=pltpu.CompilerParams(use_tc_tiling_on_sc=False),
        out_type=jax.ShapeDtypeStruct((_NC, n_nodes, feat), jnp.float32),
        scratch_types=[
            pltpu.VMEM((1, _K), jnp.int32),
            pltpu.VMEM((1, _K), jnp.int32),
            pltpu.VMEM((_K, feat), jnp.float32),
            pltpu.VMEM_SHARED((n_nodes, feat), jnp.float32),
            pltpu.VMEM((rpt, feat), jnp.float32),
            pltpu.VMEM((max(tail, 8), feat), jnp.float32),
            pltpu.SemaphoreType.DMA,
        ],
    )
    def scat_kernel(src_hbm, dst_hbm, tbl_hbm, zeros_hbm, out_hbm,
                    srcb, dstb, rows, acc, wb, wbt, sem):
        c = lax.axis_index("c")
        s = lax.axis_index("s")
        r0 = pl.multiple_of(s * rpt, 8)
        t0 = rpt * _NS
        pltpu.sync_copy(zeros_hbm.at[pl.ds(r0, rpt)], wb)
        pltpu.sync_copy(wb, acc.at[pl.ds(r0, rpt)])
        if tail:
            @pl.when(s == _NS - 1)
            def _():
                pltpu.sync_copy(zeros_hbm.at[pl.ds(t0, tail)], wbt)
                pltpu.sync_copy(wbt, acc.at[pl.ds(t0, tail)])
        plsc.subcore_barrier()

        base = (c * _NS + s) * epw

        def body(i, carry):
            off = pl.multiple_of(base + i * _K, 8)
            pltpu.sync_copy(src_hbm.at[pl.ds(off, _K)], srcb.at[0])
            pltpu.sync_copy(dst_hbm.at[pl.ds(off, _K)], dstb.at[0])
            pltpu.async_copy(tbl_hbm.at[srcb.at[0]], rows, sem).wait()
            pltpu.sync_copy(rows, acc.at[dstb.at[0]], add=True)
            return carry

        lax.fori_loop(0, chunks, body, 0)
        plsc.subcore_barrier()

        pltpu.sync_copy(acc.at[pl.ds(r0, rpt)], wb)
        pltpu.sync_copy(wb, out_hbm.at[c, pl.ds(r0, rpt)])
        if tail:
            @pl.when(s == _NS - 1)
            def _():
                pltpu.sync_copy(acc.at[pl.ds(t0, tail)], wbt)
                pltpu.sync_copy(wbt, out_hbm.at[c, pl.ds(t0, tail)])

    return scat_kernel(src, dst, rows_tbl, zeros_nf)


def _tc_first(x, w1, degp3):
    n_nodes, _ = x.shape
    h1 = w1.shape[1]

    def body(x_ref, w_ref, degp_ref, h1p_ref, dinv_ref):
        deg = degp_ref[0] + degp_ref[1] + 1.0
        dinv = lax.rsqrt(deg)
        h = jnp.dot(x_ref[...], w_ref[...], preferred_element_type=jnp.float32)
        h1p_ref[...] = h * dinv
        dinv_ref[...] = dinv

    return pl.pallas_call(
        body,
        out_shape=(
            jax.ShapeDtypeStruct((n_nodes, h1), jnp.float32),
            jax.ShapeDtypeStruct((n_nodes, 1), jnp.float32),
        ),
    )(x, w1, degp3)


def _tc_mid(s1p, h1p, dinv, b1r, w2):
    n_nodes, h1 = h1p.shape
    h2 = w2.shape[1]

    def body(s1p_ref, h1p_ref, dinv_ref, b1_ref, w2_ref, h2p_ref):
        ssum = s1p_ref[0] + s1p_ref[1] + h1p_ref[...]
        dinv = dinv_ref[...]
        act = jnp.maximum(dinv * ssum + b1_ref[...], 0.0)
        h = jnp.dot(act, w2_ref[...], preferred_element_type=jnp.float32)
        h2p_ref[...] = h * dinv

    return pl.pallas_call(
        body,
        out_shape=jax.ShapeDtypeStruct((n_nodes, h2), jnp.float32),
    )(s1p, h1p, dinv, b1r, w2)


def _tc_final(s2p, h2p, dinv, b2r, bi_row, wfc, bfcr):
    n_nodes, h2 = h2p.shape
    n_cls = wfc.shape[1]

    def body(s2p_ref, h2p_ref, dinv_ref, b2_ref, bi_ref, wfc_ref, bfc_ref,
             out_ref):
        ssum = s2p_ref[0] + s2p_ref[1] + h2p_ref[...]
        act = jnp.maximum(dinv_ref[...] * ssum + b2_ref[...], 0.0)
        gid = lax.broadcasted_iota(jnp.int32, (_G, n_nodes), 0)
        onehot = (gid == bi_ref[...]).astype(jnp.float32)
        sums = jnp.dot(onehot, act, preferred_element_type=jnp.float32)
        counts = jnp.sum(onehot, axis=1, keepdims=True)
        pooled = sums / jnp.maximum(counts, 1.0)
        logits = jnp.dot(pooled, wfc_ref[...],
                         preferred_element_type=jnp.float32) + bfc_ref[...]
        m = jnp.max(logits, axis=1, keepdims=True)
        e = jnp.exp(logits - m)
        out_ref[...] = e / jnp.sum(e, axis=1, keepdims=True)

    return pl.pallas_call(
        body,
        out_shape=jax.ShapeDtypeStruct((_G, n_cls), jnp.float32),
    )(s2p, h2p, dinv, b2r, bi_row, wfc, bfcr)


def kernel(x, edge_index, batch_index, W1, b1, W2, b2, Wfc, bfc):
    n_nodes, _ = x.shape
    h1 = W1.shape[1]
    h2 = W2.shape[1]
    src = edge_index[0]
    dst = edge_index[1]

    degp = _sc_degree(dst, jnp.zeros((n_nodes,), jnp.float32))
    h1p, dinv = _tc_first(x, W1, degp.reshape(_NC, n_nodes, 1))
    s1p = _sc_scatter(src, dst, h1p, jnp.zeros((n_nodes, h1), jnp.float32))
    h2p = _tc_mid(s1p, h1p, dinv, b1.reshape(1, -1), W2)
    s2p = _sc_scatter(src, dst, h2p, jnp.zeros((n_nodes, h2), jnp.float32))
    return _tc_final(s2p, h2p, dinv, b2.reshape(1, -1),
                     batch_index.reshape(1, -1), Wfc, bfc.reshape(1, -1))


# speedup vs baseline: 51.4186x; 3.1589x over previous
import functools

import jax
import jax.numpy as jnp
from jax import lax
from jax.experimental import pallas as pl
from jax.experimental.pallas import tpu as pltpu
from jax.experimental.pallas import tpu_sc as plsc

_NC = 2
_NS = 16
_G = 64
_K = 80


def _sc_mesh():
    return plsc.VectorSubcoreMesh(core_axis_name="c", subcore_axis_name="s")


def _sc_degree(dst2d, zeros_n):
    (n_nodes,) = zeros_n.shape
    idx_rows, _ = dst2d.shape
    chunks = idx_rows // (_NC * _NS)

    @functools.partial(
        pl.kernel,
        mesh=_sc_mesh(),
        compiler_params=pltpu.CompilerParams(use_tc_tiling_on_sc=False),
        out_type=jax.ShapeDtypeStruct((_NC * n_nodes,), jnp.float32),
        scratch_types=[
            pltpu.VMEM((chunks, _K), jnp.int32),
            pltpu.VMEM((_K,), jnp.float32),
            pltpu.VMEM_SHARED((n_nodes,), jnp.float32),
            pltpu.VMEM((n_nodes,), jnp.float32),
            pltpu.SemaphoreType.DMA,
        ],
    )
    def deg_kernel(dst_hbm, zeros_hbm, out_hbm, dstall, onesb, acc, wb, sem):
        c = lax.axis_index("c")
        s = lax.axis_index("s")
        for j in range(_K // 16):
            onesb[pl.ds(j * 16, 16)] = jnp.ones((16,), jnp.float32)

        row0 = pl.multiple_of((c * _NS + s) * chunks, 8)
        pltpu.sync_copy(dst_hbm.at[pl.ds(row0, chunks)], dstall)

        @pl.when(s == 0)
        def _():
            pltpu.sync_copy(zeros_hbm, wb)
            pltpu.sync_copy(wb, acc)

        plsc.subcore_barrier()

        def body(i, carry):
            pltpu.async_copy(onesb, acc.at[dstall.at[i]], sem, add=True)
            return carry

        lax.fori_loop(0, chunks, body, 0)
        pltpu.make_async_copy(dst_hbm.at[pl.ds(row0, chunks)], dstall, sem).wait()
        plsc.subcore_barrier()

        @pl.when(s == 0)
        def _():
            pltpu.sync_copy(acc, wb)
            pltpu.sync_copy(wb, out_hbm.at[pl.ds(pl.multiple_of(c * n_nodes, 8), n_nodes)])

    return deg_kernel(dst2d, zeros_n)


def _sc_scatter(src2d, dst2d, rows_tbl, zeros_nf):
    idx_rows, _ = src2d.shape
    n_nodes, feat = rows_tbl.shape
    chunks = idx_rows // (_NC * _NS)
    M = 5
    NB = 3
    rpt = (n_nodes // _NS) // 8 * 8
    tail = n_nodes - rpt * _NS

    @functools.partial(
        pl.kernel,
        mesh=_sc_mesh(),
        compiler_params=pltpu.CompilerParams(use_tc_tiling_on_sc=False),
        out_type=jax.ShapeDtypeStruct((_NC, n_nodes, feat), jnp.float32),
        scratch_types=[
            pltpu.VMEM((chunks, _K), jnp.int32),
            pltpu.VMEM((chunks, _K), jnp.int32),
            pltpu.VMEM((M, _K, feat), jnp.float32),
            pltpu.VMEM_SHARED((n_nodes, feat), jnp.float32),
            pltpu.VMEM((rpt, feat), jnp.float32),
            pltpu.VMEM((max(tail, 8), feat), jnp.float32),
            pltpu.SemaphoreType.DMA((M,)),
            pltpu.SemaphoreType.DMA((M,)),
        ],
    )
    def scat_kernel(src_hbm, dst_hbm, tbl_hbm, zeros_hbm, out_hbm,
                    srcall, dstall, rows, acc, wb, wbt, gsem, ssem):
        c = lax.axis_index("c")
        s = lax.axis_index("s")
        row0 = pl.multiple_of((c * _NS + s) * chunks, 8)
        pltpu.sync_copy(src_hbm.at[pl.ds(row0, chunks)], srcall)
        pltpu.sync_copy(dst_hbm.at[pl.ds(row0, chunks)], dstall)

        r0 = pl.multiple_of(s * rpt, 8)
        t0 = rpt * _NS
        pltpu.sync_copy(zeros_hbm.at[pl.ds(r0, rpt)], wb)
        pltpu.sync_copy(wb, acc.at[pl.ds(r0, rpt)])
        if tail:
            @pl.when(s == _NS - 1)
            def _():
                pltpu.sync_copy(zeros_hbm.at[pl.ds(t0, tail)], wbt)
                pltpu.sync_copy(wbt, acc.at[pl.ds(t0, tail)])
        plsc.subcore_barrier()

        def _dummy_wait(b, sem):
            pltpu.make_async_copy(
                tbl_hbm.at[pl.ds(0, _K)], rows.at[b], sem.at[b]).wait()

        for b in range(NB):
            pltpu.async_copy(tbl_hbm.at[srcall.at[b]], rows.at[b], gsem.at[b])

        def outer(o, carry):
            for b in range(M):
                ch = o * M + b
                _dummy_wait(b, gsem)
                pltpu.async_copy(rows.at[b], acc.at[dstall.at[ch]],
                                 ssem.at[b], add=True)
                nxt = ch + NB
                bp = (b + NB) % M

                @pl.when(nxt < chunks)
                def _():
                    @pl.when(ch >= M - NB)
                    def _():
                        _dummy_wait(bp, ssem)
                    pltpu.async_copy(tbl_hbm.at[srcall.at[nxt]],
                                     rows.at[bp], gsem.at[bp])
            return carry

        lax.fori_loop(0, chunks // M, outer, 0)
        for b in range(M):
            _dummy_wait(b, ssem)
        plsc.subcore_barrier()

        pltpu.sync_copy(acc.at[pl.ds(r0, rpt)], wb)
        pltpu.sync_copy(wb, out_hbm.at[c, pl.ds(r0, rpt)])
        if tail:
            @pl.when(s == _NS - 1)
            def _():
                pltpu.sync_copy(acc.at[pl.ds(t0, tail)], wbt)
                pltpu.sync_copy(wbt, out_hbm.at[c, pl.ds(t0, tail)])

    return scat_kernel(src2d, dst2d, rows_tbl, zeros_nf)


def _tc_first(x, w1, degp3):
    n_nodes, _ = x.shape
    h1 = w1.shape[1]

    def body(x_ref, w_ref, degp_ref, h1p_ref, dinv_ref):
        deg = degp_ref[0] + degp_ref[1] + 1.0
        dinv = lax.rsqrt(deg)
        h = jnp.dot(x_ref[...], w_ref[...], preferred_element_type=jnp.float32)
        h1p_ref[...] = h * dinv
        dinv_ref[...] = dinv

    return pl.pallas_call(
        body,
        out_shape=(
            jax.ShapeDtypeStruct((n_nodes, h1), jnp.float32),
            jax.ShapeDtypeStruct((n_nodes, 1), jnp.float32),
        ),
    )(x, w1, degp3)


def _tc_mid(s1p, h1p, dinv, b1r, w2):
    n_nodes, h1 = h1p.shape
    h2 = w2.shape[1]

    def body(s1p_ref, h1p_ref, dinv_ref, b1_ref, w2_ref, h2p_ref):
        ssum = s1p_ref[0] + s1p_ref[1] + h1p_ref[...]
        dinv = dinv_ref[...]
        act = jnp.maximum(dinv * ssum + b1_ref[...], 0.0)
        h = jnp.dot(act, w2_ref[...], preferred_element_type=jnp.float32)
        h2p_ref[...] = h * dinv

    return pl.pallas_call(
        body,
        out_shape=jax.ShapeDtypeStruct((n_nodes, h2), jnp.float32),
    )(s1p, h1p, dinv, b1r, w2)


def _tc_final(s2p, h2p, dinv, b2r, bi_row, wfc, bfcr):
    n_nodes, h2 = h2p.shape
    n_cls = wfc.shape[1]

    def body(s2p_ref, h2p_ref, dinv_ref, b2_ref, bi_ref, wfc_ref, bfc_ref,
             out_ref):
        ssum = s2p_ref[0] + s2p_ref[1] + h2p_ref[...]
        act = jnp.maximum(dinv_ref[...] * ssum + b2_ref[...], 0.0)
        gid = lax.broadcasted_iota(jnp.int32, (_G, n_nodes), 0)
        onehot = (gid == bi_ref[...]).astype(jnp.float32)
        sums = jnp.dot(onehot, act, preferred_element_type=jnp.float32)
        counts = jnp.sum(onehot, axis=1, keepdims=True)
        pooled = sums / jnp.maximum(counts, 1.0)
        logits = jnp.dot(pooled, wfc_ref[...],
                         preferred_element_type=jnp.float32) + bfc_ref[...]
        m = jnp.max(logits, axis=1, keepdims=True)
        e = jnp.exp(logits - m)
        out_ref[...] = e / jnp.sum(e, axis=1, keepdims=True)

    return pl.pallas_call(
        body,
        out_shape=jax.ShapeDtypeStruct((_G, n_cls), jnp.float32),
    )(s2p, h2p, dinv, b2r, bi_row, wfc, bfcr)


def kernel(x, edge_index, batch_index, W1, b1, W2, b2, Wfc, bfc):
    n_nodes, _ = x.shape
    h1 = W1.shape[1]
    h2 = W2.shape[1]
    src2d = edge_index[0].reshape(-1, _K)
    dst2d = edge_index[1].reshape(-1, _K)

    degp = _sc_degree(dst2d, jnp.zeros((n_nodes,), jnp.float32))
    h1p, dinv = _tc_first(x, W1, degp.reshape(_NC, n_nodes, 1))
    s1p = _sc_scatter(src2d, dst2d, h1p, jnp.zeros((n_nodes, h1), jnp.float32))
    h2p = _tc_mid(s1p, h1p, dinv, b1.reshape(1, -1), W2)
    s2p = _sc_scatter(src2d, dst2d, h2p, jnp.zeros((n_nodes, h2), jnp.float32))
    return _tc_final(s2p, h2p, dinv, b2.reshape(1, -1),
                     batch_index.reshape(1, -1), Wfc, bfc.reshape(1, -1))


# speedup vs baseline: 60.5689x; 1.1780x over previous
import functools

import jax
import jax.numpy as jnp
from jax import lax
from jax.experimental import pallas as pl
from jax.experimental.pallas import tpu as pltpu
from jax.experimental.pallas import tpu_sc as plsc

_NC = 2
_NS = 16
_G = 64
_K = 80


def _sc_mesh():
    return plsc.VectorSubcoreMesh(core_axis_name="c", subcore_axis_name="s")


def _sc_degree(edges3, zeros_n, n_nodes):
    _, idx_rows, _ = edges3.shape
    chunks = idx_rows // (_NC * _NS)

    @functools.partial(
        pl.kernel,
        mesh=_sc_mesh(),
        compiler_params=pltpu.CompilerParams(use_tc_tiling_on_sc=False),
        out_type=jax.ShapeDtypeStruct((_NC * n_nodes,), jnp.float32),
        scratch_types=[
            pltpu.VMEM((chunks, _K), jnp.int32),
            pltpu.VMEM((_K,), jnp.float32),
            pltpu.VMEM_SHARED((n_nodes,), jnp.float32),
            pltpu.VMEM((n_nodes,), jnp.float32),
            pltpu.SemaphoreType.DMA,
        ],
    )
    def deg_kernel(edges_hbm, zeros_hbm, out_hbm, dstall, onesb, acc, wb, sem):
        c = lax.axis_index("c")
        s = lax.axis_index("s")
        for j in range(_K // 16):
            onesb[pl.ds(j * 16, 16)] = jnp.ones((16,), jnp.float32)

        row0 = pl.multiple_of((c * _NS + s) * chunks, 8)
        pltpu.sync_copy(edges_hbm.at[1, pl.ds(row0, chunks)], dstall)

        @pl.when(s == 0)
        def _():
            pltpu.sync_copy(zeros_hbm, wb)
            pltpu.sync_copy(wb, acc)

        plsc.subcore_barrier()

        def body(i, carry):
            pltpu.async_copy(onesb, acc.at[dstall.at[i]], sem, add=True)
            return carry

        lax.fori_loop(0, chunks, body, 0)
        pltpu.make_async_copy(edges_hbm.at[1, pl.ds(row0, chunks)], dstall, sem).wait()
        plsc.subcore_barrier()

        @pl.when(s == 0)
        def _():
            pltpu.sync_copy(acc, wb)
            pltpu.sync_copy(wb, out_hbm.at[pl.ds(pl.multiple_of(c * n_nodes, 8), n_nodes)])

    return deg_kernel(edges3, zeros_n)


def _sc_scatter(edges3, rows_tbl, zeros_nf):
    _, idx_rows, _ = edges3.shape
    n_nodes, feat = rows_tbl.shape
    chunks = idx_rows // (_NC * _NS)
    M = 5
    NB = 3
    rpt = (n_nodes // _NS) // 8 * 8
    tail = n_nodes - rpt * _NS

    @functools.partial(
        pl.kernel,
        mesh=_sc_mesh(),
        compiler_params=pltpu.CompilerParams(use_tc_tiling_on_sc=False),
        out_type=jax.ShapeDtypeStruct((_NC, n_nodes, feat), jnp.float32),
        scratch_types=[
            pltpu.VMEM((chunks, _K), jnp.int32),
            pltpu.VMEM((chunks, _K), jnp.int32),
            pltpu.VMEM((M, _K, feat), jnp.float32),
            pltpu.VMEM_SHARED((n_nodes, feat), jnp.float32),
            pltpu.VMEM((rpt, feat), jnp.float32),
            pltpu.VMEM((max(tail, 8), feat), jnp.float32),
            pltpu.SemaphoreType.DMA((M,)),
            pltpu.SemaphoreType.DMA((M,)),
        ],
    )
    def scat_kernel(edges_hbm, tbl_hbm, zeros_hbm, out_hbm,
                    srcall, dstall, rows, acc, wb, wbt, gsem, ssem):
        c = lax.axis_index("c")
        s = lax.axis_index("s")
        row0 = pl.multiple_of((c * _NS + s) * chunks, 8)
        pltpu.sync_copy(edges_hbm.at[0, pl.ds(row0, chunks)], srcall)
        pltpu.sync_copy(edges_hbm.at[1, pl.ds(row0, chunks)], dstall)

        r0 = pl.multiple_of(s * rpt, 8)
        t0 = rpt * _NS
        pltpu.sync_copy(zeros_hbm.at[pl.ds(r0, rpt)], wb)
        pltpu.sync_copy(wb, acc.at[pl.ds(r0, rpt)])
        if tail:
            @pl.when(s == _NS - 1)
            def _():
                pltpu.sync_copy(zeros_hbm.at[pl.ds(t0, tail)], wbt)
                pltpu.sync_copy(wbt, acc.at[pl.ds(t0, tail)])
        plsc.subcore_barrier()

        def _dummy_wait(b, sem):
            pltpu.make_async_copy(
                tbl_hbm.at[pl.ds(0, _K)], rows.at[b], sem.at[b]).wait()

        for b in range(NB):
            pltpu.async_copy(tbl_hbm.at[srcall.at[b]], rows.at[b], gsem.at[b])

        def outer(o, carry):
            for b in range(M):
                ch = o * M + b
                _dummy_wait(b, gsem)
                pltpu.async_copy(rows.at[b], acc.at[dstall.at[ch]],
                                 ssem.at[b], add=True)
                nxt = ch + NB
                bp = (b + NB) % M

                @pl.when(nxt < chunks)
                def _():
                    @pl.when(ch >= M - NB)
                    def _():
                        _dummy_wait(bp, ssem)
                    pltpu.async_copy(tbl_hbm.at[srcall.at[nxt]],
                                     rows.at[bp], gsem.at[bp])
            return carry

        lax.fori_loop(0, chunks // M, outer, 0)
        for b in range(M):
            _dummy_wait(b, ssem)
        plsc.subcore_barrier()

        pltpu.sync_copy(acc.at[pl.ds(r0, rpt)], wb)
        pltpu.sync_copy(wb, out_hbm.at[c, pl.ds(r0, rpt)])
        if tail:
            @pl.when(s == _NS - 1)
            def _():
                pltpu.sync_copy(acc.at[pl.ds(t0, tail)], wbt)
                pltpu.sync_copy(wbt, out_hbm.at[c, pl.ds(t0, tail)])

    return scat_kernel(edges3, rows_tbl, zeros_nf)


def _col(v, n):
    return v.reshape(n, 1)


def _tc_first(x, w1, degp_flat):
    n_nodes, _ = x.shape
    h1 = w1.shape[1]

    def body(x_ref, w_ref, dp_ref, h1p_ref, dinv_ref):
        dp = dp_ref[...]
        deg = dp[:n_nodes] + dp[n_nodes:] + 1.0
        dinv = lax.rsqrt(deg)
        dinv_ref[...] = dinv
        h = jnp.dot(x_ref[...], w_ref[...], preferred_element_type=jnp.float32)
        h1p_ref[...] = h * _col(dinv, n_nodes)

    return pl.pallas_call(
        body,
        out_shape=(
            jax.ShapeDtypeStruct((n_nodes, h1), jnp.float32),
            jax.ShapeDtypeStruct((n_nodes,), jnp.float32),
        ),
    )(x, w1, degp_flat)


def _tc_mid(s1p, h1p, dinv, b1r, w2):
    n_nodes, h1 = h1p.shape
    h2 = w2.shape[1]

    def body(s1p_ref, h1p_ref, dinv_ref, b1_ref, w2_ref, h2p_ref):
        ssum = s1p_ref[0] + s1p_ref[1] + h1p_ref[...]
        dinv = _col(dinv_ref[...], n_nodes)
        act = jnp.maximum(dinv * ssum + b1_ref[...], 0.0)
        h = jnp.dot(act, w2_ref[...], preferred_element_type=jnp.float32)
        h2p_ref[...] = h * dinv

    return pl.pallas_call(
        body,
        out_shape=jax.ShapeDtypeStruct((n_nodes, h2), jnp.float32),
    )(s1p, h1p, dinv, b1r, w2)


def _tc_final(s2p, h2p, dinv, b2r, bi_row, wfc, bfcr):
    n_nodes, h2 = h2p.shape
    n_cls = wfc.shape[1]

    def body(s2p_ref, h2p_ref, dinv_ref, b2_ref, bi_ref, wfc_ref, bfc_ref,
             out_ref):
        ssum = s2p_ref[0] + s2p_ref[1] + h2p_ref[...]
        act = jnp.maximum(_col(dinv_ref[...], n_nodes) * ssum + b2_ref[...], 0.0)
        gid = lax.broadcasted_iota(jnp.int32, (_G, n_nodes), 0)
        onehot = (gid == bi_ref[...]).astype(jnp.float32)
        sums = jnp.dot(onehot, act, preferred_element_type=jnp.float32)
        counts = jnp.sum(onehot, axis=1, keepdims=True)
        pooled = sums / jnp.maximum(counts, 1.0)
        logits = jnp.dot(pooled, wfc_ref[...],
                         preferred_element_type=jnp.float32) + bfc_ref[...]
        m = jnp.max(logits, axis=1, keepdims=True)
        e = jnp.exp(logits - m)
        out_ref[...] = e / jnp.sum(e, axis=1, keepdims=True)

    return pl.pallas_call(
        body,
        out_shape=jax.ShapeDtypeStruct((_G, n_cls), jnp.float32),
    )(s2p, h2p, dinv, b2r, bi_row, wfc, bfcr)


def kernel(x, edge_index, batch_index, W1, b1, W2, b2, Wfc, bfc):
    n_nodes, _ = x.shape
    h1 = W1.shape[1]
    h2 = W2.shape[1]
    edges3 = edge_index.reshape(2, -1, _K)

    degp = _sc_degree(edges3, jnp.zeros((n_nodes,), jnp.float32), n_nodes)
    h1p, dinv = _tc_first(x, W1, degp)
    s1p = _sc_scatter(edges3, h1p, jnp.zeros((n_nodes, h1), jnp.float32))
    h2p = _tc_mid(s1p, h1p, dinv, b1.reshape(1, -1), W2)
    s2p = _sc_scatter(edges3, h2p, jnp.zeros((n_nodes, h2), jnp.float32))
    return _tc_final(s2p, h2p, dinv, b2.reshape(1, -1),
                     batch_index.reshape(1, -1), Wfc, bfc.reshape(1, -1))


# speedup vs baseline: 64.1092x; 1.0585x over previous
import functools

import jax
import jax.numpy as jnp
from jax import lax
from jax.experimental import pallas as pl
from jax.experimental.pallas import tpu as pltpu
from jax.experimental.pallas import tpu_sc as plsc

_NC = 2
_NS = 16
_G = 64
_K = 128
_M = 8
_NB = 4


def _sc_mesh():
    return plsc.VectorSubcoreMesh(core_axis_name="c", subcore_axis_name="s")


def _sc_degree(edges3, zeros_p, npad):
    _, idx_rows, _ = edges3.shape
    chunks = idx_rows // (_NC * _NS)

    @functools.partial(
        pl.kernel,
        mesh=_sc_mesh(),
        compiler_params=pltpu.CompilerParams(use_tc_tiling_on_sc=False),
        out_type=jax.ShapeDtypeStruct((_NC * npad,), jnp.float32),
        scratch_types=[
            pltpu.VMEM((chunks, _K), jnp.int32),
            pltpu.VMEM((_K,), jnp.float32),
            pltpu.VMEM_SHARED((npad,), jnp.float32),
            pltpu.VMEM((npad,), jnp.float32),
            pltpu.SemaphoreType.DMA,
        ],
    )
    def deg_kernel(edges_hbm, zeros_hbm, out_hbm, dstall, onesb, acc, wb, sem):
        c = lax.axis_index("c")
        s = lax.axis_index("s")
        for j in range(_K // 16):
            onesb[pl.ds(j * 16, 16)] = jnp.ones((16,), jnp.float32)

        row0 = pl.multiple_of((c * _NS + s) * chunks, 8)
        pltpu.sync_copy(edges_hbm.at[1, pl.ds(row0, chunks)], dstall)

        @pl.when(s == 0)
        def _():
            pltpu.sync_copy(zeros_hbm, wb)
            pltpu.sync_copy(wb, acc)

        plsc.subcore_barrier()

        def body(i, carry):
            pltpu.async_copy(onesb, acc.at[dstall.at[i]], sem, add=True)
            return carry

        lax.fori_loop(0, chunks, body, 0)
        pltpu.make_async_copy(
            edges_hbm.at[1, pl.ds(row0, chunks)], dstall, sem).wait()
        plsc.subcore_barrier()

        @pl.when(s == 0)
        def _():
            pltpu.sync_copy(acc, wb)
            pltpu.sync_copy(wb, out_hbm.at[pl.ds(pl.multiple_of(c * npad, 8),
                                                 npad)])

    return deg_kernel(edges3, zeros_p)


def _sc_scatter(edges3, rows_tbl, zeros_pf, npad):
    _, idx_rows, _ = edges3.shape
    n_nodes, feat = rows_tbl.shape
    chunks = idx_rows // (_NC * _NS)
    rpt = npad // _NS

    @functools.partial(
        pl.kernel,
        mesh=_sc_mesh(),
        compiler_params=pltpu.CompilerParams(use_tc_tiling_on_sc=False),
        out_type=jax.ShapeDtypeStruct((_NC, npad, feat), jnp.float32),
        scratch_types=[
            pltpu.VMEM((chunks, _K), jnp.int32),
            pltpu.VMEM((chunks, _K), jnp.int32),
            pltpu.VMEM((_M, _K, feat), jnp.float32),
            pltpu.VMEM_SHARED((npad, feat), jnp.float32),
            pltpu.VMEM((rpt, feat), jnp.float32),
            pltpu.SemaphoreType.DMA((_M,)),
            pltpu.SemaphoreType.DMA((_M,)),
        ],
    )
    def scat_kernel(edges_hbm, tbl_hbm, zeros_hbm, out_hbm,
                    srcall, dstall, rows, acc, wb, gsem, ssem):
        c = lax.axis_index("c")
        s = lax.axis_index("s")
        row0 = pl.multiple_of((c * _NS + s) * chunks, 8)
        pltpu.sync_copy(edges_hbm.at[0, pl.ds(row0, chunks)], srcall)
        pltpu.sync_copy(edges_hbm.at[1, pl.ds(row0, chunks)], dstall)

        r0 = pl.multiple_of(s * rpt, 8)
        pltpu.sync_copy(zeros_hbm.at[pl.ds(r0, rpt)], wb)
        pltpu.sync_copy(wb, acc.at[pl.ds(r0, rpt)])
        plsc.subcore_barrier()

        def _dummy_wait(b, sem):
            pltpu.make_async_copy(
                tbl_hbm.at[pl.ds(0, _K)], rows.at[b], sem.at[b]).wait()

        for b in range(_NB):
            pltpu.async_copy(tbl_hbm.at[srcall.at[b]], rows.at[b], gsem.at[b])

        def outer(o, carry):
            for b in range(_M):
                ch = o * _M + b
                _dummy_wait(b, gsem)
                pltpu.async_copy(rows.at[b], acc.at[dstall.at[ch]],
                                 ssem.at[b], add=True)
                nxt = ch + _NB
                bp = (b + _NB) % _M

                @pl.when(nxt < chunks)
                def _():
                    @pl.when(ch >= _M - _NB)
                    def _():
                        _dummy_wait(bp, ssem)
                    pltpu.async_copy(tbl_hbm.at[srcall.at[nxt]],
                                     rows.at[bp], gsem.at[bp])
            return carry

        lax.fori_loop(0, chunks // _M, outer, 0)
        for b in range(_M):
            _dummy_wait(b, ssem)
        plsc.subcore_barrier()

        pltpu.sync_copy(acc.at[pl.ds(r0, rpt)], wb)
        pltpu.sync_copy(wb, out_hbm.at[c, pl.ds(r0, rpt)])

    return scat_kernel(edges3, rows_tbl, zeros_pf)


def _col(v, n):
    return v.reshape(n, 1)


def _tc_first(x, w1, degp_flat, npad):
    n_nodes, _ = x.shape
    h1 = w1.shape[1]

    def body(x_ref, w_ref, dp_ref, h1p_ref, dinv_ref):
        dp = dp_ref[...]
        deg = dp[:n_nodes] + dp[npad:npad + n_nodes] + 1.0
        dinv = lax.rsqrt(deg)
        dinv_ref[...] = dinv
        h = jnp.dot(x_ref[...], w_ref[...], preferred_element_type=jnp.float32)
        h1p_ref[...] = h * _col(dinv, n_nodes)

    return pl.pallas_call(
        body,
        out_shape=(
            jax.ShapeDtypeStruct((n_nodes, h1), jnp.float32),
            jax.ShapeDtypeStruct((n_nodes,), jnp.float32),
        ),
    )(x, w1, degp_flat)


def _tc_mid(s1p, h1p, dinv, b1r, w2):
    n_nodes, h1 = h1p.shape
    h2 = w2.shape[1]

    def body(s1p_ref, h1p_ref, dinv_ref, b1_ref, w2_ref, h2p_ref):
        ssum = (s1p_ref[0, :n_nodes] + s1p_ref[1, :n_nodes] + h1p_ref[...])
        dinv = _col(dinv_ref[...], n_nodes)
        act = jnp.maximum(dinv * ssum + b1_ref[...], 0.0)
        h = jnp.dot(act, w2_ref[...], preferred_element_type=jnp.float32)
        h2p_ref[...] = h * dinv

    return pl.pallas_call(
        body,
        out_shape=jax.ShapeDtypeStruct((n_nodes, h2), jnp.float32),
    )(s1p, h1p, dinv, b1r, w2)


def _tc_final(s2p, h2p, dinv, b2r, bi, wfc, bfcr):
    n_nodes, h2 = h2p.shape
    n_cls = wfc.shape[1]

    def body(s2p_ref, h2p_ref, dinv_ref, b2_ref, bi_ref, wfc_ref, bfc_ref,
             out_ref):
        ssum = (s2p_ref[0, :n_nodes] + s2p_ref[1, :n_nodes] + h2p_ref[...])
        act = jnp.maximum(_col(dinv_ref[...], n_nodes) * ssum + b2_ref[...],
                          0.0)
        gid = lax.broadcasted_iota(jnp.int32, (_G, n_nodes), 0)
        onehot = (gid == bi_ref[...].reshape(1, n_nodes)).astype(jnp.float32)
        sums = jnp.dot(onehot, act, preferred_element_type=jnp.float32)
        counts = jnp.sum(onehot, axis=1, keepdims=True)
        pooled = sums / jnp.maximum(counts, 1.0)
        logits = jnp.dot(pooled, wfc_ref[...],
                         preferred_element_type=jnp.float32) + bfc_ref[...]
        m = jnp.max(logits, axis=1, keepdims=True)
        e = jnp.exp(logits - m)
        out_ref[...] = e / jnp.sum(e, axis=1, keepdims=True)

    return pl.pallas_call(
        body,
        out_shape=jax.ShapeDtypeStruct((_G, n_cls), jnp.float32),
    )(s2p, h2p, dinv, b2r, bi, wfc, bfcr)


def kernel(x, edge_index, batch_index, W1, b1, W2, b2, Wfc, bfc):
    n_nodes, _ = x.shape
    h1 = W1.shape[1]
    h2 = W2.shape[1]
    n_edges = edge_index.shape[1]

    npad = ((n_nodes // _NS) // 8 + (1 if n_nodes % (_NS * 8) else 0)) * 8 * _NS
    group = _NC * _NS * _M * _K
    e_pad = -(-n_edges // group) * group
    pe = e_pad - n_edges
    pad_src = (jnp.arange(pe, dtype=jnp.int32) * 131) % n_nodes
    pad_dst = n_nodes + jnp.arange(pe, dtype=jnp.int32) % (npad - n_nodes)
    src_p = jnp.concatenate([edge_index[0], pad_src])
    dst_p = jnp.concatenate([edge_index[1], pad_dst])
    edges3 = jnp.stack([src_p, dst_p]).reshape(2, e_pad // _K, _K)

    degp = _sc_degree(edges3, jnp.zeros((npad,), jnp.float32), npad)
    h1p, dinv = _tc_first(x, W1, degp, npad)
    s1p = _sc_scatter(edges3, h1p, jnp.zeros((npad, h1), jnp.float32), npad)
    h2p = _tc_mid(s1p, h1p, dinv, b1.reshape(1, -1), W2)
    s2p = _sc_scatter(edges3, h2p, jnp.zeros((npad, h2), jnp.float32), npad)
    return _tc_final(s2p, h2p, dinv, b2.reshape(1, -1),
                     batch_index, Wfc, bfc.reshape(1, -1))


# speedup vs baseline: 67.6564x; 1.0553x over previous
import functools

import jax
import jax.numpy as jnp
from jax import lax
from jax.experimental import pallas as pl
from jax.experimental.pallas import tpu as pltpu
from jax.experimental.pallas import tpu_sc as plsc

_NC = 2
_NS = 16
_G = 64
_K = 128
_M = 8
_NB = 4


def _sc_mesh():
    return plsc.VectorSubcoreMesh(core_axis_name="c", subcore_axis_name="s")


def _sc_degree(edges3, zeros_p, npad):
    _, idx_rows, _ = edges3.shape
    chunks = idx_rows // (_NC * _NS)

    @functools.partial(
        pl.kernel,
        mesh=_sc_mesh(),
        compiler_params=pltpu.CompilerParams(use_tc_tiling_on_sc=False),
        out_type=jax.ShapeDtypeStruct((_NC * npad,), jnp.float32),
        scratch_types=[
            pltpu.VMEM((chunks, _K), jnp.int32),
            pltpu.VMEM((_K,), jnp.float32),
            pltpu.VMEM_SHARED((npad,), jnp.float32),
            pltpu.VMEM((npad,), jnp.float32),
            pltpu.SemaphoreType.DMA,
        ],
    )
    def deg_kernel(edges_hbm, zeros_hbm, out_hbm, dstall, onesb, acc, wb, sem):
        c = lax.axis_index("c")
        s = lax.axis_index("s")
        for j in range(_K // 16):
            onesb[pl.ds(j * 16, 16)] = jnp.ones((16,), jnp.float32)

        row0 = pl.multiple_of((c * _NS + s) * chunks, 8)
        pltpu.sync_copy(edges_hbm.at[1, pl.ds(row0, chunks)], dstall)

        @pl.when(s == 0)
        def _():
            pltpu.sync_copy(zeros_hbm, wb)
            pltpu.sync_copy(wb, acc)

        plsc.subcore_barrier()

        def body(i, carry):
            pltpu.async_copy(onesb, acc.at[dstall.at[i]], sem, add=True)
            return carry

        lax.fori_loop(0, chunks, body, 0)
        pltpu.make_async_copy(
            edges_hbm.at[1, pl.ds(row0, chunks)], dstall, sem).wait()
        plsc.subcore_barrier()

        @pl.when(s == 0)
        def _():
            pltpu.sync_copy(acc, wb)
            pltpu.sync_copy(wb, out_hbm.at[pl.ds(pl.multiple_of(c * npad, 8),
                                                 npad)])

    return deg_kernel(edges3, zeros_p)


def _sc_scatter(edges3, rows_tbl, zeros_pf, npad):
    _, idx_rows, _ = edges3.shape
    n_nodes, feat = rows_tbl.shape
    chunks = idx_rows // (_NC * _NS)
    rpt = npad // _NS

    @functools.partial(
        pl.kernel,
        mesh=_sc_mesh(),
        compiler_params=pltpu.CompilerParams(use_tc_tiling_on_sc=False),
        out_type=jax.ShapeDtypeStruct((_NC, npad, feat), jnp.float32),
        scratch_types=[
            pltpu.VMEM((chunks, _K), jnp.int32),
            pltpu.VMEM((chunks, _K), jnp.int32),
            pltpu.VMEM((_M, _K, feat), jnp.float32),
            pltpu.VMEM_SHARED((npad, feat), jnp.float32),
            pltpu.VMEM((rpt, feat), jnp.float32),
            pltpu.SemaphoreType.DMA((_M,)),
            pltpu.SemaphoreType.DMA((_M,)),
        ],
    )
    def scat_kernel(edges_hbm, tbl_hbm, zeros_hbm, out_hbm,
                    srcall, dstall, rows, acc, wb, gsem, ssem):
        c = lax.axis_index("c")
        s = lax.axis_index("s")
        row0 = pl.multiple_of((c * _NS + s) * chunks, 8)
        pltpu.sync_copy(edges_hbm.at[0, pl.ds(row0, chunks)], srcall)
        pltpu.sync_copy(edges_hbm.at[1, pl.ds(row0, chunks)], dstall)

        r0 = pl.multiple_of(s * rpt, 8)
        pltpu.sync_copy(zeros_hbm.at[pl.ds(r0, rpt)], wb)
        pltpu.sync_copy(wb, acc.at[pl.ds(r0, rpt)])
        plsc.subcore_barrier()

        def _dummy_wait(b, sem):
            pltpu.make_async_copy(
                tbl_hbm.at[pl.ds(0, _K)], rows.at[b], sem.at[b]).wait()

        for b in range(_NB):
            pltpu.async_copy(tbl_hbm.at[srcall.at[b]], rows.at[b], gsem.at[b])

        def outer(o, carry):
            for b in range(_M):
                ch = o * _M + b
                _dummy_wait(b, gsem)
                pltpu.async_copy(rows.at[b], acc.at[dstall.at[ch]],
                                 ssem.at[b], add=True)
                nxt = ch + _NB
                bp = (b + _NB) % _M

                @pl.when(nxt < chunks)
                def _():
                    @pl.when(ch >= _M - _NB)
                    def _():
                        _dummy_wait(bp, ssem)
                    pltpu.async_copy(tbl_hbm.at[srcall.at[nxt]],
                                     rows.at[bp], gsem.at[bp])
            return carry

        lax.fori_loop(0, chunks // _M, outer, 0)
        for b in range(_M):
            _dummy_wait(b, ssem)
        plsc.subcore_barrier()

        pltpu.sync_copy(acc.at[pl.ds(r0, rpt)], wb)
        pltpu.sync_copy(wb, out_hbm.at[c, pl.ds(r0, rpt)])

    return scat_kernel(edges3, rows_tbl, zeros_pf)


def _col(v, n):
    return v.reshape(n, 1)


def _tc_first(x, w1, degp_flat, npad):
    n_nodes, _ = x.shape
    h1 = w1.shape[1]

    def body(x_ref, w_ref, dp_ref, h1p_ref, dinv_ref):
        dp = dp_ref[...]
        deg = dp[:n_nodes] + dp[npad:npad + n_nodes] + 1.0
        dinv = lax.rsqrt(deg)
        dinv_ref[...] = dinv
        h = jnp.dot(x_ref[...], w_ref[...], preferred_element_type=jnp.float32)
        h1p_ref[...] = h * _col(dinv, n_nodes)

    return pl.pallas_call(
        body,
        out_shape=(
            jax.ShapeDtypeStruct((n_nodes, h1), jnp.float32),
            jax.ShapeDtypeStruct((n_nodes,), jnp.float32),
        ),
    )(x, w1, degp_flat)


def _tc_mid(s1p, h1p, dinv, b1r, w2):
    n_nodes, h1 = h1p.shape
    h2 = w2.shape[1]

    def body(s1p_ref, h1p_ref, dinv_ref, b1_ref, w2_ref, h2p_ref):
        ssum = (s1p_ref[0, :n_nodes] + s1p_ref[1, :n_nodes] + h1p_ref[...])
        dinv = _col(dinv_ref[...], n_nodes)
        act = jnp.maximum(dinv * ssum + b1_ref[...], 0.0)
        h = jnp.dot(act, w2_ref[...], preferred_element_type=jnp.float32)
        h2p_ref[...] = h * dinv

    return pl.pallas_call(
        body,
        out_shape=jax.ShapeDtypeStruct((n_nodes, h2), jnp.float32),
    )(s1p, h1p, dinv, b1r, w2)


def _tc_final(s2p, h2p, dinv, b2r, bi, wfc, bfcr):
    n_nodes, h2 = h2p.shape
    n_cls = wfc.shape[1]

    def body(s2p_ref, h2p_ref, dinv_ref, b2_ref, bi_ref, wfc_ref, bfc_ref,
             out_ref):
        ssum = (s2p_ref[0, :n_nodes] + s2p_ref[1, :n_nodes] + h2p_ref[...])
        act = jnp.maximum(_col(dinv_ref[...], n_nodes) * ssum + b2_ref[...],
                          0.0)
        gid = lax.broadcasted_iota(jnp.int32, (_G, n_nodes), 0)
        onehot = (gid == bi_ref[...].reshape(1, n_nodes)).astype(jnp.float32)
        sums = jnp.dot(onehot, act, preferred_element_type=jnp.float32)
        counts = jnp.sum(onehot, axis=1, keepdims=True)
        pooled = sums / jnp.maximum(counts, 1.0)
        logits = jnp.dot(pooled, wfc_ref[...],
                         preferred_element_type=jnp.float32) + bfc_ref[...]
        m = jnp.max(logits, axis=1, keepdims=True)
        e = jnp.exp(logits - m)
        out_ref[...] = e / jnp.sum(e, axis=1, keepdims=True)

    return pl.pallas_call(
        body,
        out_shape=jax.ShapeDtypeStruct((_G, n_cls), jnp.float32),
    )(s2p, h2p, dinv, b2r, bi, wfc, bfcr)


def kernel(x, edge_index, batch_index, W1, b1, W2, b2, Wfc, bfc):
    n_nodes, _ = x.shape
    h1 = W1.shape[1]
    h2 = W2.shape[1]
    n_edges = edge_index.shape[1]

    npad = ((n_nodes // _NS) // 8 + (1 if n_nodes % (_NS * 8) else 0)) * 8 * _NS
    group = _NC * _NS * _M * _K
    e_pad = -(-n_edges // group) * group
    pe = e_pad - n_edges
    pad_src = (jnp.arange(pe, dtype=jnp.int32) * 131) % n_nodes
    pad_dst = n_nodes + jnp.arange(pe, dtype=jnp.int32) % (npad - n_nodes)
    pad_blk = jnp.stack([pad_src, pad_dst])
    edges3 = jnp.concatenate([edge_index, pad_blk], axis=1).reshape(
        2, e_pad // _K, _K)

    degp = _sc_degree(edges3, jnp.zeros((npad,), jnp.float32), npad)
    h1p, dinv = _tc_first(x, W1, degp, npad)
    s1p = _sc_scatter(edges3, h1p, jnp.zeros((npad, h1), jnp.float32), npad)
    h2p = _tc_mid(s1p, h1p, dinv, b1.reshape(1, -1), W2)
    s2p = _sc_scatter(edges3, h2p, jnp.zeros((npad, h2), jnp.float32), npad)
    return _tc_final(s2p, h2p, dinv, b2.reshape(1, -1),
                     batch_index, Wfc, bfc.reshape(1, -1))


# speedup vs baseline: 68.7006x; 1.0154x over previous
import functools

import jax
import jax.numpy as jnp
from jax import lax
from jax.experimental import pallas as pl
from jax.experimental.pallas import tpu as pltpu
from jax.experimental.pallas import tpu_sc as plsc

_NC = 2
_NS = 16
_G = 64
_K = 128
_M = 10
_NB = 5


def _sc_mesh():
    return plsc.VectorSubcoreMesh(core_axis_name="c", subcore_axis_name="s")


def _sc_degree(edges3, npad):
    _, idx_rows, _ = edges3.shape
    chunks = idx_rows // (_NC * _NS)

    @functools.partial(
        pl.kernel,
        mesh=_sc_mesh(),
        compiler_params=pltpu.CompilerParams(use_tc_tiling_on_sc=False),
        out_type=jax.ShapeDtypeStruct((_NC * npad,), jnp.float32),
        scratch_types=[
            pltpu.VMEM((chunks, _K), jnp.int32),
            pltpu.VMEM((_K,), jnp.float32),
            pltpu.VMEM_SHARED((npad,), jnp.float32),
            pltpu.VMEM((npad,), jnp.float32),
            pltpu.SemaphoreType.DMA,
        ],
    )
    def deg_kernel(edges_hbm, out_hbm, dstall, onesb, acc, wb, sem):
        c = lax.axis_index("c")
        s = lax.axis_index("s")
        for j in range(_K // 16):
            onesb[pl.ds(j * 16, 16)] = jnp.ones((16,), jnp.float32)

        row0 = pl.multiple_of((c * _NS + s) * chunks, 8)
        pltpu.sync_copy(edges_hbm.at[1, pl.ds(row0, chunks)], dstall)

        @pl.when(s == 0)
        def _():
            def zbody(i, carry):
                wb[pl.ds(i * 16, 16)] = jnp.zeros((16,), jnp.float32)
                return carry

            lax.fori_loop(0, npad // 16, zbody, 0)
            pltpu.sync_copy(wb, acc)

        plsc.subcore_barrier()

        def body(i, carry):
            pltpu.async_copy(onesb, acc.at[dstall.at[i]], sem, add=True)
            return carry

        lax.fori_loop(0, chunks, body, 0)
        pltpu.make_async_copy(
            edges_hbm.at[1, pl.ds(row0, chunks)], dstall, sem).wait()
        plsc.subcore_barrier()

        @pl.when(s == 0)
        def _():
            pltpu.sync_copy(acc, wb)
            pltpu.sync_copy(wb, out_hbm.at[pl.ds(pl.multiple_of(c * npad, 8),
                                                 npad)])

    return deg_kernel(edges3)


def _sc_scatter(edges3, rows_tbl, npad):
    _, idx_rows, _ = edges3.shape
    n_nodes, feat = rows_tbl.shape
    chunks = idx_rows // (_NC * _NS)
    rpt = npad // _NS

    @functools.partial(
        pl.kernel,
        mesh=_sc_mesh(),
        compiler_params=pltpu.CompilerParams(use_tc_tiling_on_sc=False),
        out_type=jax.ShapeDtypeStruct((_NC, npad, feat), jnp.float32),
        scratch_types=[
            pltpu.VMEM((chunks, _K), jnp.int32),
            pltpu.VMEM((chunks, _K), jnp.int32),
            pltpu.VMEM((_M, _K, feat), jnp.float32),
            pltpu.VMEM_SHARED((npad, feat), jnp.float32),
            pltpu.VMEM((rpt, feat), jnp.float32),
            pltpu.SemaphoreType.DMA((_M,)),
            pltpu.SemaphoreType.DMA((_M,)),
        ],
    )
    def scat_kernel(edges_hbm, tbl_hbm, out_hbm,
                    srcall, dstall, rows, acc, wb, gsem, ssem):
        c = lax.axis_index("c")
        s = lax.axis_index("s")
        row0 = pl.multiple_of((c * _NS + s) * chunks, 8)
        pltpu.sync_copy(edges_hbm.at[0, pl.ds(row0, chunks)], srcall)
        pltpu.sync_copy(edges_hbm.at[1, pl.ds(row0, chunks)], dstall)

        r0 = pl.multiple_of(s * rpt, 8)

        def zbody(i, carry):
            for j in range(feat // 16):
                wb[i, pl.ds(j * 16, 16)] = jnp.zeros((16,), jnp.float32)
            return carry

        lax.fori_loop(0, rpt, zbody, 0)
        pltpu.sync_copy(wb, acc.at[pl.ds(r0, rpt)])
        plsc.subcore_barrier()

        def _dummy_wait(b, sem):
            pltpu.make_async_copy(
                tbl_hbm.at[pl.ds(0, _K)], rows.at[b], sem.at[b]).wait()

        for b in range(_NB):
            pltpu.async_copy(tbl_hbm.at[srcall.at[b]], rows.at[b], gsem.at[b])

        def outer(o, carry):
            for b in range(_M):
                ch = o * _M + b
                _dummy_wait(b, gsem)
                pltpu.async_copy(rows.at[b], acc.at[dstall.at[ch]],
                                 ssem.at[b], add=True)
                nxt = ch + _NB
                bp = (b + _NB) % _M

                @pl.when(nxt < chunks)
                def _():
                    @pl.when(ch >= _M - _NB)
                    def _():
                        _dummy_wait(bp, ssem)
                    pltpu.async_copy(tbl_hbm.at[srcall.at[nxt]],
                                     rows.at[bp], gsem.at[bp])
            return carry

        lax.fori_loop(0, chunks // _M, outer, 0)
        for b in range(_M):
            _dummy_wait(b, ssem)
        plsc.subcore_barrier()

        pltpu.sync_copy(acc.at[pl.ds(r0, rpt)], wb)
        pltpu.sync_copy(wb, out_hbm.at[c, pl.ds(r0, rpt)])

    return scat_kernel(edges3, rows_tbl)


def _col(v, n):
    return v.reshape(n, 1)


def _tc_first(x, w1, degp_flat, npad):
    n_nodes, _ = x.shape
    h1 = w1.shape[1]

    def body(x_ref, w_ref, dp_ref, h1p_ref, dinv_ref):
        dp = dp_ref[...]
        deg = dp[:n_nodes] + dp[npad:npad + n_nodes] + 1.0
        dinv = lax.rsqrt(deg)
        dinv_ref[...] = dinv
        h = jnp.dot(x_ref[...], w_ref[...], preferred_element_type=jnp.float32)
        h1p_ref[...] = h * _col(dinv, n_nodes)

    return pl.pallas_call(
        body,
        out_shape=(
            jax.ShapeDtypeStruct((n_nodes, h1), jnp.float32),
            jax.ShapeDtypeStruct((n_nodes,), jnp.float32),
        ),
    )(x, w1, degp_flat)


def _tc_mid(s1p, h1p, dinv, b1r, w2):
    n_nodes, h1 = h1p.shape
    h2 = w2.shape[1]

    def body(s1p_ref, h1p_ref, dinv_ref, b1_ref, w2_ref, h2p_ref):
        ssum = (s1p_ref[0, :n_nodes] + s1p_ref[1, :n_nodes] + h1p_ref[...])
        dinv = _col(dinv_ref[...], n_nodes)
        act = jnp.maximum(dinv * ssum + b1_ref[...], 0.0)
        h = jnp.dot(act, w2_ref[...], preferred_element_type=jnp.float32)
        h2p_ref[...] = h * dinv

    return pl.pallas_call(
        body,
        out_shape=jax.ShapeDtypeStruct((n_nodes, h2), jnp.float32),
    )(s1p, h1p, dinv, b1r, w2)


def _tc_final(s2p, h2p, dinv, b2r, bi, wfc, bfcr):
    n_nodes, h2 = h2p.shape
    n_cls = wfc.shape[1]

    def body(s2p_ref, h2p_ref, dinv_ref, b2_ref, bi_ref, wfc_ref, bfc_ref,
             out_ref):
        ssum = (s2p_ref[0, :n_nodes] + s2p_ref[1, :n_nodes] + h2p_ref[...])
        act = jnp.maximum(_col(dinv_ref[...], n_nodes) * ssum + b2_ref[...],
                          0.0)
        gid = lax.broadcasted_iota(jnp.int32, (_G, n_nodes), 0)
        onehot = (gid == bi_ref[...].reshape(1, n_nodes)).astype(jnp.float32)
        sums = jnp.dot(onehot, act, preferred_element_type=jnp.float32)
        counts = jnp.sum(onehot, axis=1, keepdims=True)
        pooled = sums / jnp.maximum(counts, 1.0)
        logits = jnp.dot(pooled, wfc_ref[...],
                         preferred_element_type=jnp.float32) + bfc_ref[...]
        m = jnp.max(logits, axis=1, keepdims=True)
        e = jnp.exp(logits - m)
        out_ref[...] = e / jnp.sum(e, axis=1, keepdims=True)

    return pl.pallas_call(
        body,
        out_shape=jax.ShapeDtypeStruct((_G, n_cls), jnp.float32),
    )(s2p, h2p, dinv, b2r, bi, wfc, bfcr)


def kernel(x, edge_index, batch_index, W1, b1, W2, b2, Wfc, bfc):
    n_nodes, _ = x.shape
    h1 = W1.shape[1]
    h2 = W2.shape[1]
    n_edges = edge_index.shape[1]

    npad = ((n_nodes // _NS) // 8 + (1 if n_nodes % (_NS * 8) else 0)) * 8 * _NS
    group = _NC * _NS * _M * _K
    e_pad = -(-n_edges // group) * group
    pe = e_pad - n_edges
    pad_src = (jnp.arange(pe, dtype=jnp.int32) * 131) % n_nodes
    pad_dst = n_nodes + jnp.arange(pe, dtype=jnp.int32) % (npad - n_nodes)
    pad_blk = jnp.stack([pad_src, pad_dst])
    edges3 = jnp.concatenate([edge_index, pad_blk], axis=1).reshape(
        2, e_pad // _K, _K)

    degp = _sc_degree(edges3, npad)
    h1p, dinv = _tc_first(x, W1, degp, npad)
    s1p = _sc_scatter(edges3, h1p, npad)
    h2p = _tc_mid(s1p, h1p, dinv, b1.reshape(1, -1), W2)
    s2p = _sc_scatter(edges3, h2p, npad)
    return _tc_final(s2p, h2p, dinv, b2.reshape(1, -1),
                     batch_index, Wfc, bfc.reshape(1, -1))
